# Initial kernel scaffold; baseline (speedup 1.0000x reference)
import jax
import jax.numpy as jnp
from jax import lax
from jax.experimental import pallas as pl
from jax.experimental.pallas import tpu as pltpu
from jax.experimental.pallas import tpu_sc as plsc

B = 512
G = 978
N = B * G
E = N * 16
LANES = 128
E_ROWS = E // LANES
N_TILES = 16
ROWS_PER_TILE = E_ROWS // N_TILES
CHUNK_ROWS = 326
N_CHUNKS = ROWS_PER_TILE // CHUNK_ROWS
N_PER_TILE = N // N_TILES


def _sc_segment_sum(x1, x2, edges_r, zeros_n):
    mesh = plsc.VectorSubcoreMesh(core_axis_name="c", subcore_axis_name="s")

    def body(x1_hbm, x2_hbm, edges_hbm, zeros_hbm, out_hbm,
             xsh, aggsh, srcb, dstb, valb, sem_s, sem_d):
        c = lax.axis_index("c")
        s = lax.axis_index("s")
        n0 = s * N_PER_TILE

        @pl.when(c == 0)
        def _():
            pltpu.sync_copy(x1_hbm.at[pl.ds(n0, N_PER_TILE)],
                            xsh.at[pl.ds(n0, N_PER_TILE)])

        @pl.when(c != 0)
        def _():
            pltpu.sync_copy(x2_hbm.at[pl.ds(n0, N_PER_TILE)],
                            xsh.at[pl.ds(n0, N_PER_TILE)])

        pltpu.sync_copy(zeros_hbm.at[pl.ds(n0, N_PER_TILE)],
                        aggsh.at[pl.ds(n0, N_PER_TILE)])
        plsc.subcore_barrier()

        tile_row0 = s * ROWS_PER_TILE

        def chunk(i, carry):
            row0 = tile_row0 + i * CHUNK_ROWS
            cp_s = pltpu.async_copy(edges_hbm.at[0, pl.ds(row0, CHUNK_ROWS)],
                                    srcb, sem_s)
            cp_d = pltpu.async_copy(edges_hbm.at[1, pl.ds(row0, CHUNK_ROWS)],
                                    dstb, sem_d)
            cp_s.wait()
            cp_d.wait()
            pltpu.sync_copy(xsh.at[srcb], valb)
            pltpu.sync_copy(valb, aggsh.at[dstb], add=True)
            return carry

        lax.fori_loop(0, N_CHUNKS, chunk, 0)
        plsc.subcore_barrier()
        pltpu.sync_copy(aggsh.at[pl.ds(n0, N_PER_TILE)],
                        out_hbm.at[c, pl.ds(n0, N_PER_TILE)])

    f = pl.kernel(
        body,
        out_type=jax.ShapeDtypeStruct((2, N), jnp.float32),
        mesh=mesh,
        scratch_types=[
            pltpu.VMEM_SHARED((N,), jnp.float32),
            pltpu.VMEM_SHARED((N,), jnp.float32),
            pltpu.VMEM((CHUNK_ROWS, LANES), jnp.int32),
            pltpu.VMEM((CHUNK_ROWS, LANES), jnp.int32),
            pltpu.VMEM((CHUNK_ROWS, LANES), jnp.float32),
            pltpu.SemaphoreType.DMA,
            pltpu.SemaphoreType.DMA,
        ],
    )
    return f(x1, x2, edges_r, zeros_n)


def _tc_head(agg, others, gcn_w, gcn_b, fc1_w, fc1_b, fc2_w, fc2_b,
             mlp1_w, mlp1_b, mlp2_w, mlp2_b, interpret=False):
    def body(agg_ref, oth_ref, gw_ref, gb_ref, w1_ref, b1_ref, w2_ref, b2_ref,
             m1w_ref, m1b_ref, m2w_ref, m2b_ref, out_ref):
        gw = gw_ref[...]
        gb = gb_ref[...]

        def enc(a):
            h = jnp.maximum(a * gw + gb, 0.0)
            h = lax.dot_general(h, w1_ref[...], (((1,), (1,)), ((), ())),
                                preferred_element_type=jnp.float32)
            h = jnp.maximum(h + b1_ref[...], 0.0)
            o = lax.dot_general(h, w2_ref[...], (((1,), (1,)), ((), ())),
                                preferred_element_type=jnp.float32)
            return o + b2_ref[...]

        o1 = enc(agg_ref[0])
        o2 = enc(agg_ref[1])
        p1 = o1 - jnp.mean(o1, axis=1, keepdims=True)
        p2 = o2 - jnp.mean(o2, axis=1, keepdims=True)
        n1 = jnp.sum(p1 * p1, axis=1, keepdims=True)
        n2 = jnp.sum(p2 * p2, axis=1, keepdims=True)
        p12 = jnp.sum(p1 * p2, axis=1, keepdims=True)
        r = p12 / jnp.sqrt(n1 * n2)
        r2 = r * r
        cat = jnp.concatenate([r2, oth_ref[...]], axis=1)
        z = lax.dot_general(cat, m1w_ref[...], (((1,), (1,)), ((), ())),
                            preferred_element_type=jnp.float32)
        z = jnp.maximum(z + m1b_ref[...], 0.0)
        out = lax.dot_general(z, m2w_ref[...], (((1,), (1,)), ((), ())),
                              preferred_element_type=jnp.float32)
        out_ref[...] = out + m2b_ref[...]

    return pl.pallas_call(
        body,
        out_shape=jax.ShapeDtypeStruct((B, 2), jnp.float32),
        interpret=interpret,
    )(agg, others, gcn_w, gcn_b, fc1_w, fc1_b, fc2_w, fc2_b,
      mlp1_w, mlp1_b, mlp2_w, mlp2_b)


def kernel(input1, input2, edges, input_others, gcn_w, gcn_b,
           fc1_w, fc1_b, fc2_w, fc2_b, mlp1_w, mlp1_b, mlp2_w, mlp2_b):
    x1 = input1.reshape(-1)
    x2 = input2.reshape(-1)
    edges_r = edges.reshape(2, E_ROWS, LANES)
    zeros_n = jnp.zeros((N,), jnp.float32)
    agg = _sc_segment_sum(x1, x2, edges_r, zeros_n)
    return _tc_head(agg.reshape(2, B, G), input_others,
                    gcn_w, gcn_b.reshape(1, 1),
                    fc1_w, fc1_b.reshape(1, -1),
                    fc2_w, fc2_b.reshape(1, -1),
                    mlp1_w, mlp1_b.reshape(1, -1),
                    mlp2_w, mlp2_b.reshape(1, -1))


# speedup vs baseline: 409.7137x; 409.7137x over previous
import jax
import jax.numpy as jnp
from jax import lax
from jax.experimental import pallas as pl
from jax.experimental.pallas import tpu as pltpu
from jax.experimental.pallas import tpu_sc as plsc

B = 512
G = 978
N = B * G
E = N * 16
N_TILES = 16
E_PER_TILE = E // N_TILES
CHUNK = 20864
N_CHUNKS = E_PER_TILE // CHUNK
N_PER_TILE = N // N_TILES
STAGE = N_PER_TILE // 2


def _sc_segment_sum(x1, x2, edges_r, zeros_n):
    mesh = plsc.VectorSubcoreMesh(core_axis_name="c", subcore_axis_name="s")

    def body(x1_hbm, x2_hbm, edges_hbm, zeros_hbm, out_hbm,
             xsh, aggsh, srcb, dstb, valb, sem_s, sem_d):
        c = lax.axis_index("c")
        s = lax.axis_index("s")
        n0 = s * N_PER_TILE

        stage = valb.at[pl.ds(0, STAGE)]
        for k in range(N_PER_TILE // STAGE):
            p0 = n0 + k * STAGE

            @pl.when(c == 0)
            def _():
                pltpu.sync_copy(x1_hbm.at[pl.ds(p0, STAGE)], stage)

            @pl.when(c != 0)
            def _():
                pltpu.sync_copy(x2_hbm.at[pl.ds(p0, STAGE)], stage)

            pltpu.sync_copy(stage, xsh.at[pl.ds(p0, STAGE)])
            pltpu.sync_copy(zeros_hbm.at[pl.ds(p0, STAGE)], stage)
            pltpu.sync_copy(stage, aggsh.at[pl.ds(p0, STAGE)])
        plsc.subcore_barrier()

        tile_e0 = s * E_PER_TILE

        def chunk(i, carry):
            e0 = tile_e0 + i * CHUNK
            cp_s = pltpu.async_copy(edges_hbm.at[0, pl.ds(e0, CHUNK)],
                                    srcb, sem_s)
            cp_d = pltpu.async_copy(edges_hbm.at[1, pl.ds(e0, CHUNK)],
                                    dstb, sem_d)
            cp_s.wait()
            cp_d.wait()
            pltpu.sync_copy(xsh.at[srcb], valb)
            pltpu.sync_copy(valb, aggsh.at[dstb], add=True)
            return carry

        lax.fori_loop(0, N_CHUNKS, chunk, 0)
        plsc.subcore_barrier()
        for k in range(N_PER_TILE // STAGE):
            p0 = n0 + k * STAGE
            pltpu.sync_copy(aggsh.at[pl.ds(p0, STAGE)], stage)
            pltpu.sync_copy(stage, out_hbm.at[pl.ds(c * N + p0, STAGE)])

    f = pl.kernel(
        body,
        out_type=jax.ShapeDtypeStruct((2 * N,), jnp.float32),
        mesh=mesh,
        scratch_types=[
            pltpu.VMEM_SHARED((N,), jnp.float32),
            pltpu.VMEM_SHARED((N,), jnp.float32),
            pltpu.VMEM((CHUNK,), jnp.int32),
            pltpu.VMEM((CHUNK,), jnp.int32),
            pltpu.VMEM((CHUNK,), jnp.float32),
            pltpu.SemaphoreType.DMA,
            pltpu.SemaphoreType.DMA,
        ],
    )
    return f(x1, x2, edges_r, zeros_n)


def _tc_head(agg, others, gcn_w, gcn_b, fc1_w, fc1_b, fc2_w, fc2_b,
             mlp1_w, mlp1_b, mlp2_w, mlp2_b, interpret=False):
    def body(agg_ref, oth_ref, gw_ref, gb_ref, w1_ref, b1_ref, w2_ref, b2_ref,
             m1w_ref, m1b_ref, m2w_ref, m2b_ref, out_ref):
        gw = gw_ref[...]
        gb = gb_ref[...]

        def enc(a):
            h = jnp.maximum(a * gw + gb, 0.0)
            h = lax.dot_general(h, w1_ref[...], (((1,), (1,)), ((), ())),
                                preferred_element_type=jnp.float32)
            h = jnp.maximum(h + b1_ref[...], 0.0)
            o = lax.dot_general(h, w2_ref[...], (((1,), (1,)), ((), ())),
                                preferred_element_type=jnp.float32)
            return o + b2_ref[...]

        o1 = enc(agg_ref[0])
        o2 = enc(agg_ref[1])
        p1 = o1 - jnp.mean(o1, axis=1, keepdims=True)
        p2 = o2 - jnp.mean(o2, axis=1, keepdims=True)
        n1 = jnp.sum(p1 * p1, axis=1, keepdims=True)
        n2 = jnp.sum(p2 * p2, axis=1, keepdims=True)
        p12 = jnp.sum(p1 * p2, axis=1, keepdims=True)
        r = p12 / jnp.sqrt(n1 * n2)
        r2 = r * r
        cat = jnp.concatenate([r2, oth_ref[...]], axis=1)
        z = lax.dot_general(cat, m1w_ref[...], (((1,), (1,)), ((), ())),
                            preferred_element_type=jnp.float32)
        z = jnp.maximum(z + m1b_ref[...], 0.0)
        out = lax.dot_general(z, m2w_ref[...], (((1,), (1,)), ((), ())),
                              preferred_element_type=jnp.float32)
        out_ref[...] = out + m2b_ref[...]

    return pl.pallas_call(
        body,
        out_shape=jax.ShapeDtypeStruct((B, 2), jnp.float32),
        interpret=interpret,
    )(agg, others, gcn_w, gcn_b, fc1_w, fc1_b, fc2_w, fc2_b,
      mlp1_w, mlp1_b, mlp2_w, mlp2_b)


def kernel(input1, input2, edges, input_others, gcn_w, gcn_b,
           fc1_w, fc1_b, fc2_w, fc2_b, mlp1_w, mlp1_b, mlp2_w, mlp2_b):
    x1 = input1.reshape(-1)
    x2 = input2.reshape(-1)
    zeros_n = jnp.zeros((N,), jnp.float32)
    agg = _sc_segment_sum(x1, x2, edges, zeros_n)
    return _tc_head(agg.reshape(2, B, G), input_others,
                    gcn_w, gcn_b.reshape(1, 1),
                    fc1_w, fc1_b.reshape(1, -1),
                    fc2_w, fc2_b.reshape(1, -1),
                    mlp1_w, mlp1_b.reshape(1, -1),
                    mlp2_w, mlp2_b.reshape(1, -1))
